# Initial kernel scaffold; baseline (speedup 1.0000x reference)
import jax
import jax.numpy as jnp
from jax.experimental import pallas as pl


def kernel(xyz, x, rgb):
    raise NotImplementedError("write your pallas kernel here")


# speedup vs baseline: 1.5552x; 1.5552x over previous
import functools

import jax
import jax.numpy as jnp
from jax import lax
from jax.experimental import pallas as pl
from jax.experimental.pallas import tpu as pltpu

GROUP = 512
KNN = 32
LANE = 128


def _fps_body(xs_ref, ys_ref, zs_ref, idx_ref, cx_ref, cy_ref, cz_ref):
    B, S, L = xs_ref.shape
    N = S * L
    GS = GROUP // LANE
    xs = xs_ref[...]
    ys = ys_ref[...]
    zs = zs_ref[...]
    iota_n = (
        lax.broadcasted_iota(jnp.int32, (B, S, L), 1) * L
        + lax.broadcasted_iota(jnp.int32, (B, S, L), 2)
    )
    iota_g = (
        lax.broadcasted_iota(jnp.int32, (B, GS, L), 1) * L
        + lax.broadcasted_iota(jnp.int32, (B, GS, L), 2)
    )

    def body(i, state):
        distance, farthest, acc_idx, acc_cx, acc_cy, acc_cz = state
        m = iota_n == farthest
        cx = jnp.sum(jnp.where(m, xs, 0.0), axis=(1, 2), keepdims=True)
        cy = jnp.sum(jnp.where(m, ys, 0.0), axis=(1, 2), keepdims=True)
        cz = jnp.sum(jnp.where(m, zs, 0.0), axis=(1, 2), keepdims=True)
        sel = iota_g == i
        acc_idx = jnp.where(sel, farthest, acc_idx)
        acc_cx = jnp.where(sel, cx, acc_cx)
        acc_cy = jnp.where(sel, cy, acc_cy)
        acc_cz = jnp.where(sel, cz, acc_cz)
        dx = xs - cx
        dy = ys - cy
        dz = zs - cz
        dist = (dx * dx + dy * dy) + dz * dz
        distance = jnp.minimum(distance, dist)
        mx = jnp.max(distance, axis=(1, 2), keepdims=True)
        farthest = jnp.min(
            jnp.where(distance == mx, iota_n, N), axis=(1, 2), keepdims=True
        )
        return distance, farthest, acc_idx, acc_cx, acc_cy, acc_cz

    distance0 = jnp.full((B, S, L), 1e10, dtype=jnp.float32)
    farthest0 = jnp.zeros((B, 1, 1), dtype=jnp.int32)
    acc0 = (
        jnp.zeros((B, GS, L), jnp.int32),
        jnp.zeros((B, GS, L), jnp.float32),
        jnp.zeros((B, GS, L), jnp.float32),
        jnp.zeros((B, GS, L), jnp.float32),
    )
    _, _, acc_idx, acc_cx, acc_cy, acc_cz = lax.fori_loop(
        0, GROUP, body, (distance0, farthest0) + acc0
    )
    idx_ref[...] = acc_idx
    cx_ref[...] = acc_cx
    cy_ref[...] = acc_cy
    cz_ref[...] = acc_cz


def _fps(xyz):
    B, N, _ = xyz.shape
    S = N // LANE
    GS = GROUP // LANE
    xs = xyz[:, :, 0].reshape(B, S, LANE)
    ys = xyz[:, :, 1].reshape(B, S, LANE)
    zs = xyz[:, :, 2].reshape(B, S, LANE)
    out_shapes = (
        jax.ShapeDtypeStruct((B, GS, LANE), jnp.int32),
        jax.ShapeDtypeStruct((B, GS, LANE), jnp.float32),
        jax.ShapeDtypeStruct((B, GS, LANE), jnp.float32),
        jax.ShapeDtypeStruct((B, GS, LANE), jnp.float32),
    )
    idx, cx, cy, cz = pl.pallas_call(
        _fps_body,
        out_shape=out_shapes,
    )(xs, ys, zs)
    return (
        idx.reshape(B, GROUP),
        cx.reshape(B, GROUP),
        cy.reshape(B, GROUP),
        cz.reshape(B, GROUP),
    )


def kernel(xyz, x, rgb):
    fps_idx, cx, cy, cz = _fps(xyz)
    lc_xyz = jnp.stack([cx, cy, cz], axis=-1)

    take = lambda arr, idx: jnp.take_along_axis(arr, idx[:, :, None], axis=1)
    lc_x = take(x, fps_idx)
    lc_rgb = take(rgb, fps_idx)

    dist = -2.0 * jnp.matmul(lc_xyz, jnp.transpose(xyz, (0, 2, 1)))
    dist = dist + jnp.sum(lc_xyz**2, -1)[:, :, None]
    dist = dist + jnp.sum(xyz**2, -1)[:, None, :]
    _, knn_idx = jax.lax.top_k(-dist, KNN)

    take2 = lambda arr, idx: jax.vmap(lambda p, i: p[i])(arr, idx)
    knn_xyz = take2(xyz, knn_idx)
    knn_x = take2(x, knn_idx)
    knn_rgb = take2(rgb, knn_idx)
    return (lc_xyz, lc_x, lc_rgb, knn_xyz, knn_x, knn_rgb)


# speedup vs baseline: 2.6752x; 1.7202x over previous
import functools

import jax
import jax.numpy as jnp
from jax import lax
from jax.experimental import pallas as pl
from jax.experimental.pallas import tpu as pltpu
from jax.experimental.pallas import tpu_sc as plsc

GROUP = 512
KNN = 32
LANE = 128
NWORK = 32
CMAX = 96
CBUF = 144


def _fps_body(xs_ref, ys_ref, zs_ref, idx_ref, cx_ref, cy_ref, cz_ref):
    B, S, L = xs_ref.shape
    N = S * L
    GS = GROUP // LANE
    xs = xs_ref[...]
    ys = ys_ref[...]
    zs = zs_ref[...]
    iota_n = (
        lax.broadcasted_iota(jnp.int32, (B, S, L), 1) * L
        + lax.broadcasted_iota(jnp.int32, (B, S, L), 2)
    )
    iota_g = (
        lax.broadcasted_iota(jnp.int32, (B, GS, L), 1) * L
        + lax.broadcasted_iota(jnp.int32, (B, GS, L), 2)
    )

    def body(i, state):
        distance, farthest, acc_idx, acc_cx, acc_cy, acc_cz = state
        m = iota_n == farthest
        cx = jnp.sum(jnp.where(m, xs, 0.0), axis=(1, 2), keepdims=True)
        cy = jnp.sum(jnp.where(m, ys, 0.0), axis=(1, 2), keepdims=True)
        cz = jnp.sum(jnp.where(m, zs, 0.0), axis=(1, 2), keepdims=True)
        sel = iota_g == i
        acc_idx = jnp.where(sel, farthest, acc_idx)
        acc_cx = jnp.where(sel, cx, acc_cx)
        acc_cy = jnp.where(sel, cy, acc_cy)
        acc_cz = jnp.where(sel, cz, acc_cz)
        dx = xs - cx
        dy = ys - cy
        dz = zs - cz
        dist = (dx * dx + dy * dy) + dz * dz
        distance = jnp.minimum(distance, dist)
        mx = jnp.max(distance, axis=(1, 2), keepdims=True)
        farthest = jnp.min(
            jnp.where(distance == mx, iota_n, N), axis=(1, 2), keepdims=True
        )
        return distance, farthest, acc_idx, acc_cx, acc_cy, acc_cz

    distance0 = jnp.full((B, S, L), 1e10, dtype=jnp.float32)
    farthest0 = jnp.zeros((B, 1, 1), dtype=jnp.int32)
    acc0 = (
        jnp.zeros((B, GS, L), jnp.int32),
        jnp.zeros((B, GS, L), jnp.float32),
        jnp.zeros((B, GS, L), jnp.float32),
        jnp.zeros((B, GS, L), jnp.float32),
    )
    _, _, acc_idx, acc_cx, acc_cy, acc_cz = lax.fori_loop(
        0, GROUP, body, (distance0, farthest0) + acc0
    )
    idx_ref[...] = acc_idx
    cx_ref[...] = acc_cx
    cy_ref[...] = acc_cy
    cz_ref[...] = acc_cz


def _fps(xyz):
    B, N, _ = xyz.shape
    S = N // LANE
    GS = GROUP // LANE
    xs = xyz[:, :, 0].reshape(B, S, LANE)
    ys = xyz[:, :, 1].reshape(B, S, LANE)
    zs = xyz[:, :, 2].reshape(B, S, LANE)
    out_shapes = (
        jax.ShapeDtypeStruct((B, GS, LANE), jnp.int32),
        jax.ShapeDtypeStruct((B, GS, LANE), jnp.float32),
        jax.ShapeDtypeStruct((B, GS, LANE), jnp.float32),
        jax.ShapeDtypeStruct((B, GS, LANE), jnp.float32),
    )
    idx, cx, cy, cz = pl.pallas_call(
        _fps_body,
        out_shape=out_shapes,
    )(xs, ys, zs)
    return (
        idx.reshape(B, GROUP),
        cx.reshape(B, GROUP),
        cy.reshape(B, GROUP),
        cz.reshape(B, GROUP),
    )


def _dist_body(q_ref, p_ref, xs_ref, ys_ref, zs_ref, out_ref):
    q = q_ref[0]
    p = p_ref[0]
    xs = xs_ref[...][0]
    ys = ys_ref[...][0]
    zs = zs_ref[...][0]
    dot = jnp.dot(q, p, preferred_element_type=jnp.float32)
    qx = q[:, 0:1]
    qy = q[:, 1:2]
    qz = q[:, 2:3]
    qn = (qx * qx + qy * qy) + qz * qz
    pn = (xs * xs + ys * ys) + zs * zs
    out_ref[0] = ((-2.0 * dot) + qn) + pn


def _dist(lc_xyz, xyz):
    B, G, _ = lc_xyz.shape
    N = xyz.shape[1]
    xyz_t = jnp.transpose(xyz, (0, 2, 1))
    xs = xyz[:, :, 0].reshape(B, 1, N)
    ys = xyz[:, :, 1].reshape(B, 1, N)
    zs = xyz[:, :, 2].reshape(B, 1, N)
    GB = 128
    qspec = pl.BlockSpec((1, GB, 3), lambda b, g: (b, g, 0))
    pspec = pl.BlockSpec((1, 3, N), lambda b, g: (b, 0, 0))
    cspec = pl.BlockSpec((1, 1, N), lambda b, g: (b, 0, 0))
    return pl.pallas_call(
        _dist_body,
        grid=(B, G // GB),
        in_specs=[qspec, pspec, cspec, cspec, cspec],
        out_specs=pl.BlockSpec((1, GB, N), lambda b, g: (b, g, 0)),
        out_shape=jax.ShapeDtypeStruct((B, G, N), jnp.float32),
    )(lc_xyz, xyz_t, xs, ys, zs)


def _vmerge(ad, ai, bd, bi):
    brd = lax.rev(bd, (0,))
    bri = lax.rev(bi, (0,))
    mle = (ad < brd) | ((ad == brd) & (ai <= bri))
    lod = jnp.where(mle, ad, brd)
    loi = jnp.where(mle, ai, bri)
    hid = jnp.where(mle, brd, ad)
    hii = jnp.where(mle, bri, ai)
    lod, loi = plsc.sort_key_val(lod, loi)
    hid, hii = plsc.sort_key_val(hid, hii)
    return lod, loi, hid, hii


def _topk_sc(dist):
    B, G, N = dist.shape
    rows = B * G
    rpw = rows // NWORK
    flat = dist.reshape(rows, N)
    inf = jnp.float32(jnp.inf)

    mesh = plsc.VectorSubcoreMesh(core_axis_name="c", subcore_axis_name="s")

    @functools.partial(
        pl.kernel,
        out_type=jax.ShapeDtypeStruct((rows * KNN,), jnp.int32),
        mesh=mesh,
        scratch_types=[
            pltpu.VMEM((N,), jnp.float32),
            pltpu.VMEM((CBUF + 1,), jnp.float32),
            pltpu.VMEM((CBUF + 1,), jnp.int32),
            pltpu.VMEM((rpw * KNN,), jnp.int32),
        ],
        compiler_params=pltpu.CompilerParams(needs_layout_passes=False),
    )
    def k(dist_hbm, out_hbm, dbuf, cd, ci, obuf):
        wid = lax.axis_index("s") * 2 + lax.axis_index("c")
        base_row = wid * rpw
        iota16 = lax.broadcasted_iota(jnp.int32, (16,), 0)

        def consolidate(carry):
            T, cnt, r0d, r0i, r1d, r1i = carry
            cd[pl.ds(cnt, 16)] = jnp.full((16,), inf, jnp.float32)
            nv = (cnt + 15) // 16

            def merge_one(i, rc):
                r0d, r0i, r1d, r1i = rc
                sd = cd[pl.ds(i * 16, 16)]
                si = ci[pl.ds(i * 16, 16)]
                sd, si = plsc.sort_key_val(sd, si)
                l0d, l0i, h0d, h0i = _vmerge(r0d, r0i, sd, si)
                l1d, l1i, _, _ = _vmerge(r1d, r1i, h0d, h0i)
                return l0d, l0i, l1d, l1i

            r0d, r0i, r1d, r1i = lax.fori_loop(
                0, nv, merge_one, (r0d, r0i, r1d, r1i)
            )
            return jnp.max(r1d), jnp.int32(0), r0d, r0i, r1d, r1i

        def scan_step(j, carry):
            T, cnt, r0d, r0i, r1d, r1i = carry
            v = dbuf[pl.ds(j * 16, 16)]
            m = v < T
            cs = plsc.cumsum(jnp.where(m, 1, 0))
            pos = jnp.where(m, cnt + cs - 1, CBUF)
            plsc.store_scatter(cd, [pos], v)
            plsc.store_scatter(ci, [pos], iota16 + j * 16)
            cnt = cnt + jnp.max(cs)
            carry = (T, cnt, r0d, r0i, r1d, r1i)
            return lax.cond(cnt >= CMAX, consolidate, lambda c: c, carry)

        def row_fn(rl, _):
            pltpu.sync_copy(flat_ref.at[base_row + rl], dbuf)
            z16 = jnp.zeros((16,), jnp.int32)
            i16 = jnp.full((16,), inf, jnp.float32)
            carry = (inf, jnp.int32(0), i16, z16, i16, z16)
            carry = lax.fori_loop(0, N // 16, scan_step, carry)
            _, _, _, r0i, _, r1i = consolidate(carry)
            obuf[pl.ds(rl * KNN, 16)] = r0i
            obuf[pl.ds(rl * KNN + 16, 16)] = r1i
            return 0

        flat_ref = dist_hbm
        lax.fori_loop(0, rpw, row_fn, 0)
        pltpu.sync_copy(obuf, out_hbm.at[pl.ds(base_row * KNN, rpw * KNN)])

    return k(flat).reshape(B, G, KNN)


def kernel(xyz, x, rgb):
    fps_idx, cx, cy, cz = _fps(xyz)
    lc_xyz = jnp.stack([cx, cy, cz], axis=-1)

    take = lambda arr, idx: jnp.take_along_axis(arr, idx[:, :, None], axis=1)
    lc_x = take(x, fps_idx)
    lc_rgb = take(rgb, fps_idx)

    dist = _dist(lc_xyz, xyz)
    knn_idx = _topk_sc(dist)

    take2 = lambda arr, idx: jax.vmap(lambda p, i: p[i])(arr, idx)
    knn_xyz = take2(xyz, knn_idx)
    knn_x = take2(x, knn_idx)
    knn_rgb = take2(rgb, knn_idx)
    return (lc_xyz, lc_x, lc_rgb, knn_xyz, knn_x, knn_rgb)


# speedup vs baseline: 9.3811x; 3.5067x over previous
import functools

import jax
import jax.numpy as jnp
from jax import lax
from jax.experimental import pallas as pl
from jax.experimental.pallas import tpu as pltpu
from jax.experimental.pallas import tpu_sc as plsc

GROUP = 512
KNN = 32
LANE = 128
NWORK = 32
CMAX = 96
CBUF = 144


def _fps_body(xs_ref, ys_ref, zs_ref, idx_ref, cx_ref, cy_ref, cz_ref):
    B, S, L = xs_ref.shape
    N = S * L
    GS = GROUP // LANE
    xs = xs_ref[...]
    ys = ys_ref[...]
    zs = zs_ref[...]
    iota_n = (
        lax.broadcasted_iota(jnp.int32, (B, S, L), 1) * L
        + lax.broadcasted_iota(jnp.int32, (B, S, L), 2)
    )
    iota_g = (
        lax.broadcasted_iota(jnp.int32, (B, GS, L), 1) * L
        + lax.broadcasted_iota(jnp.int32, (B, GS, L), 2)
    )

    def body(i, state):
        distance, farthest, acc_idx, acc_cx, acc_cy, acc_cz = state
        m = iota_n == farthest
        cx = jnp.sum(jnp.where(m, xs, 0.0), axis=(1, 2), keepdims=True)
        cy = jnp.sum(jnp.where(m, ys, 0.0), axis=(1, 2), keepdims=True)
        cz = jnp.sum(jnp.where(m, zs, 0.0), axis=(1, 2), keepdims=True)
        sel = iota_g == i
        acc_idx = jnp.where(sel, farthest, acc_idx)
        acc_cx = jnp.where(sel, cx, acc_cx)
        acc_cy = jnp.where(sel, cy, acc_cy)
        acc_cz = jnp.where(sel, cz, acc_cz)
        dx = xs - cx
        dy = ys - cy
        dz = zs - cz
        dist = (dx * dx + dy * dy) + dz * dz
        distance = jnp.minimum(distance, dist)
        mx = jnp.max(distance, axis=(1, 2), keepdims=True)
        farthest = jnp.min(
            jnp.where(distance == mx, iota_n, N), axis=(1, 2), keepdims=True
        )
        return distance, farthest, acc_idx, acc_cx, acc_cy, acc_cz

    distance0 = jnp.full((B, S, L), 1e10, dtype=jnp.float32)
    farthest0 = jnp.zeros((B, 1, 1), dtype=jnp.int32)
    acc0 = (
        jnp.zeros((B, GS, L), jnp.int32),
        jnp.zeros((B, GS, L), jnp.float32),
        jnp.zeros((B, GS, L), jnp.float32),
        jnp.zeros((B, GS, L), jnp.float32),
    )
    _, _, acc_idx, acc_cx, acc_cy, acc_cz = lax.fori_loop(
        0, GROUP, body, (distance0, farthest0) + acc0
    )
    idx_ref[...] = acc_idx
    cx_ref[...] = acc_cx
    cy_ref[...] = acc_cy
    cz_ref[...] = acc_cz


def _fps(xyz):
    B, N, _ = xyz.shape
    S = N // LANE
    GS = GROUP // LANE
    xs = xyz[:, :, 0].reshape(B, S, LANE)
    ys = xyz[:, :, 1].reshape(B, S, LANE)
    zs = xyz[:, :, 2].reshape(B, S, LANE)
    out_shapes = (
        jax.ShapeDtypeStruct((B, GS, LANE), jnp.int32),
        jax.ShapeDtypeStruct((B, GS, LANE), jnp.float32),
        jax.ShapeDtypeStruct((B, GS, LANE), jnp.float32),
        jax.ShapeDtypeStruct((B, GS, LANE), jnp.float32),
    )
    idx, cx, cy, cz = pl.pallas_call(
        _fps_body,
        out_shape=out_shapes,
    )(xs, ys, zs)
    return (
        idx.reshape(B, GROUP),
        cx.reshape(B, GROUP),
        cy.reshape(B, GROUP),
        cz.reshape(B, GROUP),
    )


def _dist_body(q_ref, p_ref, xs_ref, ys_ref, zs_ref, out_ref):
    q = q_ref[0]
    p = p_ref[0]
    xs = xs_ref[...][0]
    ys = ys_ref[...][0]
    zs = zs_ref[...][0]
    dot = jnp.dot(q, p, preferred_element_type=jnp.float32)
    qx = q[:, 0:1]
    qy = q[:, 1:2]
    qz = q[:, 2:3]
    qn = (qx * qx + qy * qy) + qz * qz
    pn = (xs * xs + ys * ys) + zs * zs
    out_ref[0] = ((-2.0 * dot) + qn) + pn


def _dist(lc_xyz, xyz):
    B, G, _ = lc_xyz.shape
    N = xyz.shape[1]
    xyz_t = jnp.transpose(xyz, (0, 2, 1))
    xs = xyz[:, :, 0].reshape(B, 1, N)
    ys = xyz[:, :, 1].reshape(B, 1, N)
    zs = xyz[:, :, 2].reshape(B, 1, N)
    GB = 128
    qspec = pl.BlockSpec((1, GB, 3), lambda b, g: (b, g, 0))
    pspec = pl.BlockSpec((1, 3, N), lambda b, g: (b, 0, 0))
    cspec = pl.BlockSpec((1, 1, N), lambda b, g: (b, 0, 0))
    return pl.pallas_call(
        _dist_body,
        grid=(B, G // GB),
        in_specs=[qspec, pspec, cspec, cspec, cspec],
        out_specs=pl.BlockSpec((1, GB, N), lambda b, g: (b, g, 0)),
        out_shape=jax.ShapeDtypeStruct((B, G, N), jnp.float32),
    )(lc_xyz, xyz_t, xs, ys, zs)


def _vmerge(ad, ai, bd, bi):
    brd = lax.rev(bd, (0,))
    bri = lax.rev(bi, (0,))
    mle = (ad < brd) | ((ad == brd) & (ai <= bri))
    lod = jnp.where(mle, ad, brd)
    loi = jnp.where(mle, ai, bri)
    hid = jnp.where(mle, brd, ad)
    hii = jnp.where(mle, bri, ai)
    lod, loi = plsc.sort_key_val(lod, loi)
    hid, hii = plsc.sort_key_val(hid, hii)
    return lod, loi, hid, hii


def _topk_sc(dist):
    B, G, N = dist.shape
    rows = B * G
    rpw = rows // NWORK
    flat = dist.reshape(rows, N)
    inf = jnp.float32(jnp.inf)

    mesh = plsc.VectorSubcoreMesh(core_axis_name="c", subcore_axis_name="s")

    @functools.partial(
        pl.kernel,
        out_type=jax.ShapeDtypeStruct((rows * KNN,), jnp.int32),
        mesh=mesh,
        scratch_types=[
            pltpu.VMEM((N,), jnp.float32),
            pltpu.VMEM((CBUF + 1,), jnp.float32),
            pltpu.VMEM((CBUF + 1,), jnp.int32),
            pltpu.VMEM((rpw * KNN,), jnp.int32),
        ],
        compiler_params=pltpu.CompilerParams(needs_layout_passes=False),
    )
    def k(dist_hbm, out_hbm, dbuf, cd, ci, obuf):
        wid = lax.axis_index("s") * 2 + lax.axis_index("c")
        base_row = wid * rpw
        iota16 = lax.broadcasted_iota(jnp.int32, (16,), 0)

        def consolidate(carry):
            T, cnt, r0d, r0i, r1d, r1i = carry
            cd[pl.ds(cnt, 16)] = jnp.full((16,), inf, jnp.float32)
            nv = (cnt + 15) // 16

            def merge_one(i, rc):
                r0d, r0i, r1d, r1i = rc
                sd = cd[pl.ds(i * 16, 16)]
                si = ci[pl.ds(i * 16, 16)]
                sd, si = plsc.sort_key_val(sd, si)
                l0d, l0i, h0d, h0i = _vmerge(r0d, r0i, sd, si)
                l1d, l1i, _, _ = _vmerge(r1d, r1i, h0d, h0i)
                return l0d, l0i, l1d, l1i

            r0d, r0i, r1d, r1i = lax.fori_loop(
                0, nv, merge_one, (r0d, r0i, r1d, r1i)
            )
            return jnp.max(r1d), jnp.int32(0), r0d, r0i, r1d, r1i

        def scan_step(j, carry):
            T, cnt, r0d, r0i, r1d, r1i = carry
            v = dbuf[pl.ds(j * 16, 16)]
            m = v < T
            cs = plsc.cumsum(jnp.where(m, 1, 0))
            pos = jnp.where(m, cnt + cs - 1, CBUF)
            plsc.store_scatter(cd, [pos], v)
            plsc.store_scatter(ci, [pos], iota16 + j * 16)
            cnt = cnt + jnp.max(cs)
            carry = (T, cnt, r0d, r0i, r1d, r1i)
            return lax.cond(cnt >= CMAX, consolidate, lambda c: c, carry)

        def row_fn(rl, _):
            pltpu.sync_copy(flat_ref.at[base_row + rl], dbuf)
            z16 = jnp.zeros((16,), jnp.int32)
            i16 = jnp.full((16,), inf, jnp.float32)
            carry = (inf, jnp.int32(0), i16, z16, i16, z16)
            carry = lax.fori_loop(0, N // 16, scan_step, carry)
            _, _, _, r0i, _, r1i = consolidate(carry)
            obuf[pl.ds(rl * KNN, 16)] = r0i
            obuf[pl.ds(rl * KNN + 16, 16)] = r1i
            return 0

        flat_ref = dist_hbm
        lax.fori_loop(0, rpw, row_fn, 0)
        pltpu.sync_copy(obuf, out_hbm.at[pl.ds(base_row * KNN, rpw * KNN)])

    return k(flat).reshape(B, G, KNN)


def _gather_sc(x, xyz, rgb, fps_idx, knn_idx):
    B, N, D = x.shape
    G, K = GROUP, KNN
    rows = B * G
    rpw = rows // NWORK
    x_flat = x.reshape(B * N, D)
    planes = [xyz[:, :, i] for i in range(3)] + [rgb[:, :, i] for i in range(3)]
    fps_flat = fps_idx.reshape(rows)
    knn_flat = knn_idx.reshape(rows * K)

    f32 = jnp.float32
    out_type = (
        jax.ShapeDtypeStruct((rows, D), f32),
        jax.ShapeDtypeStruct((rows,), f32),
        jax.ShapeDtypeStruct((rows,), f32),
        jax.ShapeDtypeStruct((rows,), f32),
        jax.ShapeDtypeStruct((rows * K, D), f32),
        jax.ShapeDtypeStruct((rows * K,), f32),
        jax.ShapeDtypeStruct((rows * K,), f32),
        jax.ShapeDtypeStruct((rows * K,), f32),
        jax.ShapeDtypeStruct((rows * K,), f32),
        jax.ShapeDtypeStruct((rows * K,), f32),
        jax.ShapeDtypeStruct((rows * K,), f32),
    )
    mesh = plsc.VectorSubcoreMesh(core_axis_name="c", subcore_axis_name="s")

    @functools.partial(
        pl.kernel,
        out_type=out_type,
        mesh=mesh,
        scratch_types=[
            [pltpu.VMEM((N,), f32) for _ in range(6)],
            pltpu.VMEM((rpw,), jnp.int32),
            pltpu.VMEM((rpw * K,), jnp.int32),
            pltpu.VMEM((rpw * K,), jnp.int32),
            pltpu.VMEM((128,), jnp.int32),
            pltpu.VMEM((128, D), f32),
            [pltpu.VMEM((rpw,), f32) for _ in range(3)],
            [pltpu.VMEM((rpw * K,), f32) for _ in range(6)],
            pltpu.SemaphoreType.DMA,
        ],
        compiler_params=pltpu.CompilerParams(needs_layout_passes=False),
    )
    def k(
        xs_h, ys_h, zs_h, r_h, g_h, b_h, xf_h, fps_h, knn_h,
        lcx_h, lr_h, lg_h, lb_h, knnx_h, kxs_h, kys_h, kzs_h, kr_h, kg_h, kb_h,
        pv, fi, ki, kg, gbuf, rv, lst, kst, sem,
    ):
        wid = lax.axis_index("s") * 2 + lax.axis_index("c")
        b = wid // (NWORK // B)
        bbase = b * N
        plane_hs = (xs_h, ys_h, zs_h, r_h, g_h, b_h)
        for i in range(6):
            pltpu.sync_copy(plane_hs[i].at[b], pv[i])
        pltpu.sync_copy(fps_h.at[pl.ds(wid * rpw, rpw)], fi)
        pltpu.sync_copy(knn_h.at[pl.ds(wid * rpw * K, rpw * K)], ki)

        for j in range(rpw // 16):
            sl = pl.ds(j * 16, 16)
            idx16 = fi[sl]
            for i in range(3):
                lst[i][sl] = plsc.load_gather(pv[3 + i], [idx16])
            gbuf[sl] = idx16 + bbase
        pltpu.async_copy(xf_h.at[gbuf], rv, sem).wait()
        pltpu.sync_copy(rv, lcx_h.at[pl.ds(wid * rpw, rpw)])
        for i, oh in enumerate((lr_h, lg_h, lb_h)):
            pltpu.sync_copy(lst[i], oh.at[pl.ds(wid * rpw, rpw)])

        def plane_step(j, _):
            sl = pl.ds(j * 16, 16)
            idx16 = ki[sl]
            for i in range(6):
                kst[i][sl] = plsc.load_gather(pv[i], [idx16])
            kg[sl] = idx16 + bbase
            return 0

        lax.fori_loop(0, rpw * K // 16, plane_step, 0)
        outs = (kxs_h, kys_h, kzs_h, kr_h, kg_h, kb_h)
        for i in range(6):
            pltpu.sync_copy(kst[i], outs[i].at[pl.ds(wid * rpw * K, rpw * K)])

        def chunk_step(cI, _):
            for jj in range(8):
                gbuf[pl.ds(jj * 16, 16)] = kg[pl.ds(cI * 128 + jj * 16, 16)]
            pltpu.async_copy(xf_h.at[gbuf], rv, sem).wait()
            pltpu.sync_copy(
                rv, knnx_h.at[pl.ds(wid * rpw * K + cI * 128, 128)]
            )
            return 0

        lax.fori_loop(0, rpw * K // 128, chunk_step, 0)

    res = k(*planes, x_flat, fps_flat, knn_flat)
    lc_x = res[0].reshape(B, G, D)
    lc_rgb = jnp.stack(res[1:4], axis=-1).reshape(B, G, 3)
    knn_x = res[4].reshape(B, G, K, D)
    knn_xyz = jnp.stack(res[5:8], axis=-1).reshape(B, G, K, 3)
    knn_rgb = jnp.stack(res[8:11], axis=-1).reshape(B, G, K, 3)
    return lc_x, lc_rgb, knn_x, knn_xyz, knn_rgb


def kernel(xyz, x, rgb):
    fps_idx, cx, cy, cz = _fps(xyz)
    lc_xyz = jnp.stack([cx, cy, cz], axis=-1)

    dist = _dist(lc_xyz, xyz)
    knn_idx = _topk_sc(dist)

    lc_x, lc_rgb, knn_x, knn_xyz, knn_rgb = _gather_sc(
        x, xyz, rgb, fps_idx, knn_idx
    )
    return (lc_xyz, lc_x, lc_rgb, knn_xyz, knn_x, knn_rgb)


# speedup vs baseline: 14.9258x; 1.5911x over previous
import functools

import jax
import jax.numpy as jnp
from jax import lax
from jax.experimental import pallas as pl
from jax.experimental.pallas import tpu as pltpu
from jax.experimental.pallas import tpu_sc as plsc

GROUP = 512
KNN = 32
LANE = 128
NWORK = 32
CMAX = 96
CBUF = 144


def _fps_body(xs_ref, ys_ref, zs_ref, idx_ref, cx_ref, cy_ref, cz_ref):
    B, S, L = xs_ref.shape
    N = S * L
    GS = GROUP // LANE
    xs = xs_ref[...]
    ys = ys_ref[...]
    zs = zs_ref[...]
    iota_n = (
        lax.broadcasted_iota(jnp.int32, (B, S, L), 1) * L
        + lax.broadcasted_iota(jnp.int32, (B, S, L), 2)
    )
    iota_g = (
        lax.broadcasted_iota(jnp.int32, (B, GS, L), 1) * L
        + lax.broadcasted_iota(jnp.int32, (B, GS, L), 2)
    )

    def body(i, state):
        distance, farthest, acc_idx, acc_cx, acc_cy, acc_cz = state
        m = iota_n == farthest
        cx = jnp.sum(jnp.where(m, xs, 0.0), axis=(1, 2), keepdims=True)
        cy = jnp.sum(jnp.where(m, ys, 0.0), axis=(1, 2), keepdims=True)
        cz = jnp.sum(jnp.where(m, zs, 0.0), axis=(1, 2), keepdims=True)
        sel = iota_g == i
        acc_idx = jnp.where(sel, farthest, acc_idx)
        acc_cx = jnp.where(sel, cx, acc_cx)
        acc_cy = jnp.where(sel, cy, acc_cy)
        acc_cz = jnp.where(sel, cz, acc_cz)
        dx = xs - cx
        dy = ys - cy
        dz = zs - cz
        dist = (dx * dx + dy * dy) + dz * dz
        distance = jnp.minimum(distance, dist)
        mx = jnp.max(distance, axis=(1, 2), keepdims=True)
        farthest = jnp.min(
            jnp.where(distance == mx, iota_n, N), axis=(1, 2), keepdims=True
        )
        return distance, farthest, acc_idx, acc_cx, acc_cy, acc_cz

    distance0 = jnp.full((B, S, L), 1e10, dtype=jnp.float32)
    farthest0 = jnp.zeros((B, 1, 1), dtype=jnp.int32)
    acc0 = (
        jnp.zeros((B, GS, L), jnp.int32),
        jnp.zeros((B, GS, L), jnp.float32),
        jnp.zeros((B, GS, L), jnp.float32),
        jnp.zeros((B, GS, L), jnp.float32),
    )
    _, _, acc_idx, acc_cx, acc_cy, acc_cz = lax.fori_loop(
        0, GROUP, body, (distance0, farthest0) + acc0
    )
    idx_ref[...] = acc_idx
    cx_ref[...] = acc_cx
    cy_ref[...] = acc_cy
    cz_ref[...] = acc_cz


def _fps(xyz):
    B, N, _ = xyz.shape
    S = N // LANE
    GS = GROUP // LANE
    xs = xyz[:, :, 0].reshape(B, S, LANE)
    ys = xyz[:, :, 1].reshape(B, S, LANE)
    zs = xyz[:, :, 2].reshape(B, S, LANE)
    out_shapes = (
        jax.ShapeDtypeStruct((B, GS, LANE), jnp.int32),
        jax.ShapeDtypeStruct((B, GS, LANE), jnp.float32),
        jax.ShapeDtypeStruct((B, GS, LANE), jnp.float32),
        jax.ShapeDtypeStruct((B, GS, LANE), jnp.float32),
    )
    idx, cx, cy, cz = pl.pallas_call(
        _fps_body,
        out_shape=out_shapes,
    )(xs, ys, zs)
    return (
        idx.reshape(B, GROUP),
        cx.reshape(B, GROUP),
        cy.reshape(B, GROUP),
        cz.reshape(B, GROUP),
    )


def _dist_body(q_ref, p_ref, xs_ref, ys_ref, zs_ref, out_ref, cm_ref):
    q = q_ref[0]
    p = p_ref[0]
    xs = xs_ref[...][0]
    ys = ys_ref[...][0]
    zs = zs_ref[...][0]
    dot = jnp.dot(q, p, preferred_element_type=jnp.float32)
    qx = q[:, 0:1]
    qy = q[:, 1:2]
    qz = q[:, 2:3]
    qn = (qx * qx + qy * qy) + qz * qz
    pn = (xs * xs + ys * ys) + zs * zs
    d = ((-2.0 * dot) + qn) + pn
    out_ref[0] = d
    GB, N = d.shape
    cm_ref[0] = jnp.min(d.reshape(GB, N // 16, 16), axis=-1)


def _dist(lc_xyz, xyz):
    B, G, _ = lc_xyz.shape
    N = xyz.shape[1]
    xyz_t = jnp.transpose(xyz, (0, 2, 1))
    xs = xyz[:, :, 0].reshape(B, 1, N)
    ys = xyz[:, :, 1].reshape(B, 1, N)
    zs = xyz[:, :, 2].reshape(B, 1, N)
    GB = 128
    qspec = pl.BlockSpec((1, GB, 3), lambda b, g: (b, g, 0))
    pspec = pl.BlockSpec((1, 3, N), lambda b, g: (b, 0, 0))
    cspec = pl.BlockSpec((1, 1, N), lambda b, g: (b, 0, 0))
    return pl.pallas_call(
        _dist_body,
        grid=(B, G // GB),
        in_specs=[qspec, pspec, cspec, cspec, cspec],
        out_specs=(
            pl.BlockSpec((1, GB, N), lambda b, g: (b, g, 0)),
            pl.BlockSpec((1, GB, N // 16), lambda b, g: (b, g, 0)),
        ),
        out_shape=(
            jax.ShapeDtypeStruct((B, G, N), jnp.float32),
            jax.ShapeDtypeStruct((B, G, N // 16), jnp.float32),
        ),
    )(lc_xyz, xyz_t, xs, ys, zs)


def _vmerge(ad, ai, bd, bi):
    brd = lax.rev(bd, (0,))
    bri = lax.rev(bi, (0,))
    mle = (ad < brd) | ((ad == brd) & (ai <= bri))
    lod = jnp.where(mle, ad, brd)
    loi = jnp.where(mle, ai, bri)
    hid = jnp.where(mle, brd, ad)
    hii = jnp.where(mle, bri, ai)
    lod, loi = plsc.sort_key_val(lod, loi)
    hid, hii = plsc.sort_key_val(hid, hii)
    return lod, loi, hid, hii


def _topk_sc(dist, cm):
    B, G, N = dist.shape
    rows = B * G
    NC = N // 16
    rpw = rows // NWORK
    chunks = dist.reshape(rows * (N // LANE), LANE)
    cmf = cm.reshape(rows, NC)
    inf = jnp.float32(jnp.inf)

    mesh = plsc.VectorSubcoreMesh(core_axis_name="c", subcore_axis_name="s")

    @functools.partial(
        pl.kernel,
        out_type=jax.ShapeDtypeStruct((rows * KNN,), jnp.int32),
        mesh=mesh,
        scratch_types=[
            pltpu.VMEM((NC,), jnp.float32),
            pltpu.VMEM((32,), jnp.int32),
            pltpu.VMEM((32,), jnp.int32),
            pltpu.VMEM((32, LANE), jnp.float32),
            pltpu.VMEM((CBUF + 1,), jnp.float32),
            pltpu.VMEM((CBUF + 1,), jnp.int32),
            pltpu.VMEM((rpw * KNN,), jnp.int32),
            pltpu.SemaphoreType.DMA,
        ],
        compiler_params=pltpu.CompilerParams(needs_layout_passes=False),
    )
    def k(chunks_hbm, cm_hbm, out_hbm, cmbuf, cidl, cidg, cbuf, cd, ci,
          obuf, sem):
        wid = lax.axis_index("s") * 2 + lax.axis_index("c")
        base_row = wid * rpw
        iota16 = lax.broadcasted_iota(jnp.int32, (16,), 0)

        def consolidate(carry):
            T, cnt, r0d, r0i, r1d, r1i = carry
            cd[pl.ds(cnt, 16)] = jnp.full((16,), inf, jnp.float32)
            nv = (cnt + 15) // 16

            def merge_one(i, rc):
                r0d, r0i, r1d, r1i = rc
                sd = cd[pl.ds(i * 16, 16)]
                si = ci[pl.ds(i * 16, 16)]
                sd, si = plsc.sort_key_val(sd, si)
                l0d, l0i, h0d, h0i = _vmerge(r0d, r0i, sd, si)
                l1d, l1i, _, _ = _vmerge(r1d, r1i, h0d, h0i)
                return l0d, l0i, l1d, l1i

            r0d, r0i, r1d, r1i = lax.fori_loop(
                0, nv, merge_one, (r0d, r0i, r1d, r1i)
            )
            return jnp.max(r1d), jnp.int32(0), r0d, r0i, r1d, r1i

        def append(v, idx16, carry):
            T, cnt, r0d, r0i, r1d, r1i = carry
            m = v < T
            cs = plsc.cumsum(jnp.where(m, 1, 0))
            pos = jnp.where(m, cnt + cs - 1, CBUF)
            plsc.store_scatter(cd, [pos], v)
            plsc.store_scatter(ci, [pos], idx16)
            cnt = cnt + jnp.max(cs)
            carry = (T, cnt, r0d, r0i, r1d, r1i)
            return lax.cond(cnt >= CMAX, consolidate, lambda c: c, carry)

        def row_fn(rl, _):
            row = base_row + rl
            pltpu.sync_copy(cm_hbm.at[row], cmbuf)
            z16 = jnp.zeros((16,), jnp.int32)
            i16 = jnp.full((16,), inf, jnp.float32)
            carry0 = (inf, jnp.int32(0), i16, z16, i16, z16)

            def p1(j, carry):
                return append(cmbuf[pl.ds(j * 16, 16)], iota16 + j * 16,
                              carry)

            carry = lax.fori_loop(0, NC // 16, p1, carry0)
            _, _, _, c0i, _, c1i = consolidate(carry)

            a0, _ = plsc.sort_key_val(c0i, c0i)
            a1, _ = plsc.sort_key_val(c1i, c1i)
            lo, _, hi, _ = _vmerge(a0, a0, a1, a1)
            gbase = row * (N // LANE)
            cidl[pl.ds(0, 16)] = lo
            cidl[pl.ds(16, 16)] = hi
            cidg[pl.ds(0, 16)] = lo // 8 + gbase
            cidg[pl.ds(16, 16)] = hi // 8 + gbase
            pltpu.async_copy(chunks_hbm.at[cidg], cbuf, sem).wait()

            def p2(v, carry):
                cid = plsc.load_gather(cidl, [jnp.full((16,), v, jnp.int32)])
                off = jnp.max(lax.rem(cid, 8))
                d16 = cbuf[v, pl.ds(off * 16, 16)]
                return append(d16, cid * 16 + iota16, carry)

            carry = lax.fori_loop(0, 32, p2, carry0)
            _, _, _, r0i, _, r1i = consolidate(carry)
            obuf[pl.ds(rl * KNN, 16)] = r0i
            obuf[pl.ds(rl * KNN + 16, 16)] = r1i
            return 0

        lax.fori_loop(0, rpw, row_fn, 0)
        pltpu.sync_copy(obuf, out_hbm.at[pl.ds(base_row * KNN, rpw * KNN)])

    return k(chunks, cmf).reshape(B, G, KNN)


def _gather_sc(x, xyz, rgb, fps_idx, knn_idx):
    B, N, D = x.shape
    G, K = GROUP, KNN
    rows = B * G
    rpw = rows // NWORK
    x_flat = x.reshape(B * N, D)
    planes = [xyz[:, :, i] for i in range(3)] + [rgb[:, :, i] for i in range(3)]
    fps_flat = fps_idx.reshape(rows)
    knn_flat = knn_idx.reshape(rows * K)

    f32 = jnp.float32
    out_type = (
        jax.ShapeDtypeStruct((rows, D), f32),
        jax.ShapeDtypeStruct((rows,), f32),
        jax.ShapeDtypeStruct((rows,), f32),
        jax.ShapeDtypeStruct((rows,), f32),
        jax.ShapeDtypeStruct((rows * K, D), f32),
        jax.ShapeDtypeStruct((rows * K,), f32),
        jax.ShapeDtypeStruct((rows * K,), f32),
        jax.ShapeDtypeStruct((rows * K,), f32),
        jax.ShapeDtypeStruct((rows * K,), f32),
        jax.ShapeDtypeStruct((rows * K,), f32),
        jax.ShapeDtypeStruct((rows * K,), f32),
    )
    mesh = plsc.VectorSubcoreMesh(core_axis_name="c", subcore_axis_name="s")

    @functools.partial(
        pl.kernel,
        out_type=out_type,
        mesh=mesh,
        scratch_types=[
            [pltpu.VMEM((N,), f32) for _ in range(6)],
            pltpu.VMEM((rpw,), jnp.int32),
            pltpu.VMEM((rpw * K,), jnp.int32),
            pltpu.VMEM((rpw * K,), jnp.int32),
            pltpu.VMEM((128,), jnp.int32),
            pltpu.VMEM((128, D), f32),
            [pltpu.VMEM((rpw,), f32) for _ in range(3)],
            [pltpu.VMEM((rpw * K,), f32) for _ in range(6)],
            pltpu.SemaphoreType.DMA,
        ],
        compiler_params=pltpu.CompilerParams(needs_layout_passes=False),
    )
    def k(
        xs_h, ys_h, zs_h, r_h, g_h, b_h, xf_h, fps_h, knn_h,
        lcx_h, lr_h, lg_h, lb_h, knnx_h, kxs_h, kys_h, kzs_h, kr_h, kg_h, kb_h,
        pv, fi, ki, kg, gbuf, rv, lst, kst, sem,
    ):
        wid = lax.axis_index("s") * 2 + lax.axis_index("c")
        b = wid // (NWORK // B)
        bbase = b * N
        plane_hs = (xs_h, ys_h, zs_h, r_h, g_h, b_h)
        for i in range(6):
            pltpu.sync_copy(plane_hs[i].at[b], pv[i])
        pltpu.sync_copy(fps_h.at[pl.ds(wid * rpw, rpw)], fi)
        pltpu.sync_copy(knn_h.at[pl.ds(wid * rpw * K, rpw * K)], ki)

        for j in range(rpw // 16):
            sl = pl.ds(j * 16, 16)
            idx16 = fi[sl]
            for i in range(3):
                lst[i][sl] = plsc.load_gather(pv[3 + i], [idx16])
            gbuf[sl] = idx16 + bbase
        pltpu.async_copy(xf_h.at[gbuf], rv, sem).wait()
        pltpu.sync_copy(rv, lcx_h.at[pl.ds(wid * rpw, rpw)])
        for i, oh in enumerate((lr_h, lg_h, lb_h)):
            pltpu.sync_copy(lst[i], oh.at[pl.ds(wid * rpw, rpw)])

        def plane_step(j, _):
            sl = pl.ds(j * 16, 16)
            idx16 = ki[sl]
            for i in range(6):
                kst[i][sl] = plsc.load_gather(pv[i], [idx16])
            kg[sl] = idx16 + bbase
            return 0

        lax.fori_loop(0, rpw * K // 16, plane_step, 0)
        outs = (kxs_h, kys_h, kzs_h, kr_h, kg_h, kb_h)
        for i in range(6):
            pltpu.sync_copy(kst[i], outs[i].at[pl.ds(wid * rpw * K, rpw * K)])

        def chunk_step(cI, _):
            for jj in range(8):
                gbuf[pl.ds(jj * 16, 16)] = kg[pl.ds(cI * 128 + jj * 16, 16)]
            pltpu.async_copy(xf_h.at[gbuf], rv, sem).wait()
            pltpu.sync_copy(
                rv, knnx_h.at[pl.ds(wid * rpw * K + cI * 128, 128)]
            )
            return 0

        lax.fori_loop(0, rpw * K // 128, chunk_step, 0)

    res = k(*planes, x_flat, fps_flat, knn_flat)
    lc_x = res[0].reshape(B, G, D)
    lc_rgb = jnp.stack(res[1:4], axis=-1).reshape(B, G, 3)
    knn_x = res[4].reshape(B, G, K, D)
    knn_xyz = jnp.stack(res[5:8], axis=-1).reshape(B, G, K, 3)
    knn_rgb = jnp.stack(res[8:11], axis=-1).reshape(B, G, K, 3)
    return lc_x, lc_rgb, knn_x, knn_xyz, knn_rgb


def kernel(xyz, x, rgb):
    fps_idx, cx, cy, cz = _fps(xyz)
    lc_xyz = jnp.stack([cx, cy, cz], axis=-1)

    dist, cm = _dist(lc_xyz, xyz)
    knn_idx = _topk_sc(dist, cm)

    lc_x, lc_rgb, knn_x, knn_xyz, knn_rgb = _gather_sc(
        x, xyz, rgb, fps_idx, knn_idx
    )
    return (lc_xyz, lc_x, lc_rgb, knn_xyz, knn_x, knn_rgb)


# speedup vs baseline: 20.5664x; 1.3779x over previous
import functools

import jax
import jax.numpy as jnp
from jax import lax
from jax.experimental import pallas as pl
from jax.experimental.pallas import tpu as pltpu
from jax.experimental.pallas import tpu_sc as plsc

GROUP = 512
KNN = 32
LANE = 128
NWORK = 32
CMAX = 96
CBUF = 144


def _fps_body(xs_ref, ys_ref, zs_ref, idx_ref, cx_ref, cy_ref, cz_ref):
    B, S, L = xs_ref.shape
    N = S * L
    GS = GROUP // LANE
    xs = xs_ref[...]
    ys = ys_ref[...]
    zs = zs_ref[...]
    iota_n = (
        lax.broadcasted_iota(jnp.int32, (B, S, L), 1) * L
        + lax.broadcasted_iota(jnp.int32, (B, S, L), 2)
    )
    iota_g = (
        lax.broadcasted_iota(jnp.int32, (B, GS, L), 1) * L
        + lax.broadcasted_iota(jnp.int32, (B, GS, L), 2)
    )

    def body(i, state):
        distance, farthest, acc_idx, acc_cx, acc_cy, acc_cz = state
        m = iota_n == farthest
        cx = jnp.sum(jnp.where(m, xs, 0.0), axis=(1, 2), keepdims=True)
        cy = jnp.sum(jnp.where(m, ys, 0.0), axis=(1, 2), keepdims=True)
        cz = jnp.sum(jnp.where(m, zs, 0.0), axis=(1, 2), keepdims=True)
        sel = iota_g == i
        acc_idx = jnp.where(sel, farthest, acc_idx)
        acc_cx = jnp.where(sel, cx, acc_cx)
        acc_cy = jnp.where(sel, cy, acc_cy)
        acc_cz = jnp.where(sel, cz, acc_cz)
        dx = xs - cx
        dy = ys - cy
        dz = zs - cz
        dist = (dx * dx + dy * dy) + dz * dz
        distance = jnp.minimum(distance, dist)
        mx = jnp.max(distance, axis=(1, 2), keepdims=True)
        farthest = jnp.min(
            jnp.where(distance == mx, iota_n, N), axis=(1, 2), keepdims=True
        )
        return distance, farthest, acc_idx, acc_cx, acc_cy, acc_cz

    distance0 = jnp.full((B, S, L), 1e10, dtype=jnp.float32)
    farthest0 = jnp.zeros((B, 1, 1), dtype=jnp.int32)
    acc0 = (
        jnp.zeros((B, GS, L), jnp.int32),
        jnp.zeros((B, GS, L), jnp.float32),
        jnp.zeros((B, GS, L), jnp.float32),
        jnp.zeros((B, GS, L), jnp.float32),
    )
    _, _, acc_idx, acc_cx, acc_cy, acc_cz = lax.fori_loop(
        0, GROUP, body, (distance0, farthest0) + acc0
    )
    idx_ref[...] = acc_idx
    cx_ref[...] = acc_cx
    cy_ref[...] = acc_cy
    cz_ref[...] = acc_cz


def _fps(xyz):
    B, N, _ = xyz.shape
    S = N // LANE
    GS = GROUP // LANE
    xs = xyz[:, :, 0].reshape(B, S, LANE)
    ys = xyz[:, :, 1].reshape(B, S, LANE)
    zs = xyz[:, :, 2].reshape(B, S, LANE)
    out_shapes = (
        jax.ShapeDtypeStruct((B, GS, LANE), jnp.int32),
        jax.ShapeDtypeStruct((B, GS, LANE), jnp.float32),
        jax.ShapeDtypeStruct((B, GS, LANE), jnp.float32),
        jax.ShapeDtypeStruct((B, GS, LANE), jnp.float32),
    )
    idx, cx, cy, cz = pl.pallas_call(
        _fps_body,
        out_shape=out_shapes,
    )(xs, ys, zs)
    return (
        idx.reshape(B, GROUP),
        cx.reshape(B, GROUP),
        cy.reshape(B, GROUP),
        cz.reshape(B, GROUP),
    )


def _dist_body(
    q_ref, p_ref, xs_ref, ys_ref, zs_ref,
    pp_ref, xsp_ref, ysp_ref, zsp_ref, out_ref, cm_ref,
):
    q = q_ref[0]
    p = p_ref[0]
    xs = xs_ref[...][0]
    ys = ys_ref[...][0]
    zs = zs_ref[...][0]
    dot = jnp.dot(q, p, preferred_element_type=jnp.float32)
    qx = q[:, 0:1]
    qy = q[:, 1:2]
    qz = q[:, 2:3]
    qn = (qx * qx + qy * qy) + qz * qz
    pn = (xs * xs + ys * ys) + zs * zs
    d = ((-2.0 * dot) + qn) + pn
    out_ref[0] = d
    pp = pp_ref[0]
    xsp = xsp_ref[...][0]
    ysp = ysp_ref[...][0]
    zsp = zsp_ref[...][0]
    dotp = jnp.dot(q, pp, preferred_element_type=jnp.float32)
    pnp = (xsp * xsp + ysp * ysp) + zsp * zsp
    dp = ((-2.0 * dotp) + qn) + pnp
    GB, N = d.shape
    NC = N // 16
    cm = dp[:, 0:NC]
    for kk in range(1, 16):
        cm = jnp.minimum(cm, dp[:, kk * NC : (kk + 1) * NC])
    cm_ref[0] = cm


def _dist(lc_xyz, xyz):
    B, G, _ = lc_xyz.shape
    N = xyz.shape[1]
    xyz_t = jnp.transpose(xyz, (0, 2, 1))
    xs = xyz[:, :, 0].reshape(B, 1, N)
    ys = xyz[:, :, 1].reshape(B, 1, N)
    zs = xyz[:, :, 2].reshape(B, 1, N)
    permute = lambda a: (
        a.reshape(B, a.shape[1], N // 16, 16)
        .transpose(0, 1, 3, 2)
        .reshape(B, a.shape[1], N)
    )
    xyz_p = permute(xyz_t)
    xsp, ysp, zsp = permute(xs), permute(ys), permute(zs)
    GB = 128
    qspec = pl.BlockSpec((1, GB, 3), lambda b, g: (b, g, 0))
    pspec = pl.BlockSpec((1, 3, N), lambda b, g: (b, 0, 0))
    cspec = pl.BlockSpec((1, 1, N), lambda b, g: (b, 0, 0))
    return pl.pallas_call(
        _dist_body,
        grid=(B, G // GB),
        in_specs=[qspec, pspec, cspec, cspec, cspec,
                  pspec, cspec, cspec, cspec],
        out_specs=(
            pl.BlockSpec((1, GB, N), lambda b, g: (b, g, 0)),
            pl.BlockSpec((1, GB, N // 16), lambda b, g: (b, g, 0)),
        ),
        out_shape=(
            jax.ShapeDtypeStruct((B, G, N), jnp.float32),
            jax.ShapeDtypeStruct((B, G, N // 16), jnp.float32),
        ),
    )(lc_xyz, xyz_t, xs, ys, zs, xyz_p, xsp, ysp, zsp)


def _vmerge(ad, ai, bd, bi):
    brd = lax.rev(bd, (0,))
    bri = lax.rev(bi, (0,))
    mle = (ad < brd) | ((ad == brd) & (ai <= bri))
    lod = jnp.where(mle, ad, brd)
    loi = jnp.where(mle, ai, bri)
    hid = jnp.where(mle, brd, ad)
    hii = jnp.where(mle, bri, ai)
    lod, loi = plsc.sort_key_val(lod, loi)
    hid, hii = plsc.sort_key_val(hid, hii)
    return lod, loi, hid, hii


def _topk_sc(dist, cm):
    B, G, N = dist.shape
    rows = B * G
    NC = N // 16
    rpw = rows // NWORK
    chunks = dist.reshape(rows * (N // LANE), LANE)
    cmf = cm.reshape(rows, NC)
    inf = jnp.float32(jnp.inf)

    mesh = plsc.VectorSubcoreMesh(core_axis_name="c", subcore_axis_name="s")

    @functools.partial(
        pl.kernel,
        out_type=jax.ShapeDtypeStruct((rows * KNN,), jnp.int32),
        mesh=mesh,
        scratch_types=[
            pltpu.VMEM((NC,), jnp.float32),
            pltpu.VMEM((32,), jnp.int32),
            pltpu.VMEM((32,), jnp.int32),
            pltpu.VMEM((32, LANE), jnp.float32),
            pltpu.VMEM((CBUF + 1,), jnp.float32),
            pltpu.VMEM((CBUF + 1,), jnp.int32),
            pltpu.VMEM((rpw * KNN,), jnp.int32),
            pltpu.SemaphoreType.DMA,
        ],
        compiler_params=pltpu.CompilerParams(needs_layout_passes=False),
    )
    def k(chunks_hbm, cm_hbm, out_hbm, cmbuf, cidl, cidg, cbuf, cd, ci,
          obuf, sem):
        wid = lax.axis_index("s") * 2 + lax.axis_index("c")
        base_row = wid * rpw
        iota16 = lax.broadcasted_iota(jnp.int32, (16,), 0)

        def consolidate(carry):
            T, cnt, r0d, r0i, r1d, r1i = carry
            cd[pl.ds(cnt, 16)] = jnp.full((16,), inf, jnp.float32)
            nv = (cnt + 15) // 16

            def merge_one(i, rc):
                r0d, r0i, r1d, r1i = rc
                sd = cd[pl.ds(i * 16, 16)]
                si = ci[pl.ds(i * 16, 16)]
                sd, si = plsc.sort_key_val(sd, si)
                l0d, l0i, h0d, h0i = _vmerge(r0d, r0i, sd, si)
                l1d, l1i, _, _ = _vmerge(r1d, r1i, h0d, h0i)
                return l0d, l0i, l1d, l1i

            r0d, r0i, r1d, r1i = lax.fori_loop(
                0, nv, merge_one, (r0d, r0i, r1d, r1i)
            )
            return jnp.max(r1d), jnp.int32(0), r0d, r0i, r1d, r1i

        def append(v, idx16, carry):
            T, cnt, r0d, r0i, r1d, r1i = carry
            m = v < T
            cs = plsc.cumsum(jnp.where(m, 1, 0))
            pos = jnp.where(m, cnt + cs - 1, CBUF)
            plsc.store_scatter(cd, [pos], v)
            plsc.store_scatter(ci, [pos], idx16)
            cnt = cnt + jnp.max(cs)
            carry = (T, cnt, r0d, r0i, r1d, r1i)
            return lax.cond(cnt >= CMAX, consolidate, lambda c: c, carry)

        def row_fn(rl, _):
            row = base_row + rl
            pltpu.sync_copy(cm_hbm.at[row], cmbuf)
            z16 = jnp.zeros((16,), jnp.int32)
            i16 = jnp.full((16,), inf, jnp.float32)
            carry0 = (inf, jnp.int32(0), i16, z16, i16, z16)

            def p1(j, carry):
                return append(cmbuf[pl.ds(j * 16, 16)], iota16 + j * 16,
                              carry)

            carry = lax.fori_loop(0, NC // 16, p1, carry0)
            _, _, _, c0i, _, c1i = consolidate(carry)

            a0, _ = plsc.sort_key_val(c0i, c0i)
            a1, _ = plsc.sort_key_val(c1i, c1i)
            lo, _, hi, _ = _vmerge(a0, a0, a1, a1)
            gbase = row * (N // LANE)
            cidl[pl.ds(0, 16)] = lo
            cidl[pl.ds(16, 16)] = hi
            cidg[pl.ds(0, 16)] = lo // 8 + gbase
            cidg[pl.ds(16, 16)] = hi // 8 + gbase
            pltpu.async_copy(chunks_hbm.at[cidg], cbuf, sem).wait()

            def p2(v, carry):
                cid = plsc.load_gather(cidl, [jnp.full((16,), v, jnp.int32)])
                off = jnp.max(lax.rem(cid, 8))
                d16 = cbuf[v, pl.ds(off * 16, 16)]
                return append(d16, cid * 16 + iota16, carry)

            carry = lax.fori_loop(0, 32, p2, carry0)
            _, _, _, r0i, _, r1i = consolidate(carry)
            obuf[pl.ds(rl * KNN, 16)] = r0i
            obuf[pl.ds(rl * KNN + 16, 16)] = r1i
            return 0

        lax.fori_loop(0, rpw, row_fn, 0)
        pltpu.sync_copy(obuf, out_hbm.at[pl.ds(base_row * KNN, rpw * KNN)])

    return k(chunks, cmf).reshape(B, G, KNN)


def _gather_sc(x, xyz, rgb, fps_idx, knn_idx):
    B, N, D = x.shape
    G, K = GROUP, KNN
    rows = B * G
    rpw = rows // NWORK
    x_flat = x.reshape(B * N, D)
    planes = [xyz[:, :, i] for i in range(3)] + [rgb[:, :, i] for i in range(3)]
    fps_flat = fps_idx.reshape(rows)
    knn_flat = knn_idx.reshape(rows * K)

    f32 = jnp.float32
    out_type = (
        jax.ShapeDtypeStruct((rows, D), f32),
        jax.ShapeDtypeStruct((rows,), f32),
        jax.ShapeDtypeStruct((rows,), f32),
        jax.ShapeDtypeStruct((rows,), f32),
        jax.ShapeDtypeStruct((rows * K, D), f32),
        jax.ShapeDtypeStruct((rows * K,), f32),
        jax.ShapeDtypeStruct((rows * K,), f32),
        jax.ShapeDtypeStruct((rows * K,), f32),
        jax.ShapeDtypeStruct((rows * K,), f32),
        jax.ShapeDtypeStruct((rows * K,), f32),
        jax.ShapeDtypeStruct((rows * K,), f32),
    )
    mesh = plsc.VectorSubcoreMesh(core_axis_name="c", subcore_axis_name="s")

    @functools.partial(
        pl.kernel,
        out_type=out_type,
        mesh=mesh,
        scratch_types=[
            [pltpu.VMEM((N,), f32) for _ in range(6)],
            pltpu.VMEM((rpw,), jnp.int32),
            pltpu.VMEM((rpw * K,), jnp.int32),
            pltpu.VMEM((rpw * K,), jnp.int32),
            pltpu.VMEM((128,), jnp.int32),
            pltpu.VMEM((128, D), f32),
            [pltpu.VMEM((rpw,), f32) for _ in range(3)],
            [pltpu.VMEM((rpw * K,), f32) for _ in range(6)],
            pltpu.SemaphoreType.DMA,
        ],
        compiler_params=pltpu.CompilerParams(needs_layout_passes=False),
    )
    def k(
        xs_h, ys_h, zs_h, r_h, g_h, b_h, xf_h, fps_h, knn_h,
        lcx_h, lr_h, lg_h, lb_h, knnx_h, kxs_h, kys_h, kzs_h, kr_h, kg_h, kb_h,
        pv, fi, ki, kg, gbuf, rv, lst, kst, sem,
    ):
        wid = lax.axis_index("s") * 2 + lax.axis_index("c")
        b = wid // (NWORK // B)
        bbase = b * N
        plane_hs = (xs_h, ys_h, zs_h, r_h, g_h, b_h)
        for i in range(6):
            pltpu.sync_copy(plane_hs[i].at[b], pv[i])
        pltpu.sync_copy(fps_h.at[pl.ds(wid * rpw, rpw)], fi)
        pltpu.sync_copy(knn_h.at[pl.ds(wid * rpw * K, rpw * K)], ki)

        for j in range(rpw // 16):
            sl = pl.ds(j * 16, 16)
            idx16 = fi[sl]
            for i in range(3):
                lst[i][sl] = plsc.load_gather(pv[3 + i], [idx16])
            gbuf[sl] = idx16 + bbase
        pltpu.async_copy(xf_h.at[gbuf], rv, sem).wait()
        pltpu.sync_copy(rv, lcx_h.at[pl.ds(wid * rpw, rpw)])
        for i, oh in enumerate((lr_h, lg_h, lb_h)):
            pltpu.sync_copy(lst[i], oh.at[pl.ds(wid * rpw, rpw)])

        def plane_step(j, _):
            sl = pl.ds(j * 16, 16)
            idx16 = ki[sl]
            for i in range(6):
                kst[i][sl] = plsc.load_gather(pv[i], [idx16])
            kg[sl] = idx16 + bbase
            return 0

        lax.fori_loop(0, rpw * K // 16, plane_step, 0)
        outs = (kxs_h, kys_h, kzs_h, kr_h, kg_h, kb_h)
        for i in range(6):
            pltpu.sync_copy(kst[i], outs[i].at[pl.ds(wid * rpw * K, rpw * K)])

        def chunk_step(cI, _):
            for jj in range(8):
                gbuf[pl.ds(jj * 16, 16)] = kg[pl.ds(cI * 128 + jj * 16, 16)]
            pltpu.async_copy(xf_h.at[gbuf], rv, sem).wait()
            pltpu.sync_copy(
                rv, knnx_h.at[pl.ds(wid * rpw * K + cI * 128, 128)]
            )
            return 0

        lax.fori_loop(0, rpw * K // 128, chunk_step, 0)

    res = k(*planes, x_flat, fps_flat, knn_flat)
    lc_x = res[0].reshape(B, G, D)
    lc_rgb = jnp.stack(res[1:4], axis=-1).reshape(B, G, 3)
    knn_x = res[4].reshape(B, G, K, D)
    knn_xyz = jnp.stack(res[5:8], axis=-1).reshape(B, G, K, 3)
    knn_rgb = jnp.stack(res[8:11], axis=-1).reshape(B, G, K, 3)
    return lc_x, lc_rgb, knn_x, knn_xyz, knn_rgb


def kernel(xyz, x, rgb):
    fps_idx, cx, cy, cz = _fps(xyz)
    lc_xyz = jnp.stack([cx, cy, cz], axis=-1)

    dist, cm = _dist(lc_xyz, xyz)
    knn_idx = _topk_sc(dist, cm)

    lc_x, lc_rgb, knn_x, knn_xyz, knn_rgb = _gather_sc(
        x, xyz, rgb, fps_idx, knn_idx
    )
    return (lc_xyz, lc_x, lc_rgb, knn_xyz, knn_x, knn_rgb)


# speedup vs baseline: 21.3854x; 1.0398x over previous
import functools

import jax
import jax.numpy as jnp
from jax import lax
from jax.experimental import pallas as pl
from jax.experimental.pallas import tpu as pltpu
from jax.experimental.pallas import tpu_sc as plsc

GROUP = 512
KNN = 32
LANE = 128
NWORK = 32
CMAX = 96
CBUF = 144


def _fps_body(xs_ref, ys_ref, zs_ref, idx_ref, cx_ref, cy_ref, cz_ref):
    B, S, L = xs_ref.shape
    N = S * L
    GS = GROUP // LANE
    xs = xs_ref[...]
    ys = ys_ref[...]
    zs = zs_ref[...]
    iota_n = (
        lax.broadcasted_iota(jnp.int32, (B, S, L), 1) * L
        + lax.broadcasted_iota(jnp.int32, (B, S, L), 2)
    )
    iota_g = (
        lax.broadcasted_iota(jnp.int32, (B, GS, L), 1) * L
        + lax.broadcasted_iota(jnp.int32, (B, GS, L), 2)
    )

    def body(i, state):
        distance, farthest, acc_idx, acc_cx, acc_cy, acc_cz = state
        m = iota_n == farthest
        cx = jnp.sum(jnp.where(m, xs, 0.0), axis=(1, 2), keepdims=True)
        cy = jnp.sum(jnp.where(m, ys, 0.0), axis=(1, 2), keepdims=True)
        cz = jnp.sum(jnp.where(m, zs, 0.0), axis=(1, 2), keepdims=True)
        sel = iota_g == i
        acc_idx = jnp.where(sel, farthest, acc_idx)
        acc_cx = jnp.where(sel, cx, acc_cx)
        acc_cy = jnp.where(sel, cy, acc_cy)
        acc_cz = jnp.where(sel, cz, acc_cz)
        dx = xs - cx
        dy = ys - cy
        dz = zs - cz
        dist = (dx * dx + dy * dy) + dz * dz
        distance = jnp.minimum(distance, dist)
        mx = jnp.max(distance, axis=(1, 2), keepdims=True)
        farthest = jnp.min(
            jnp.where(distance == mx, iota_n, N), axis=(1, 2), keepdims=True
        )
        return distance, farthest, acc_idx, acc_cx, acc_cy, acc_cz

    distance0 = jnp.full((B, S, L), 1e10, dtype=jnp.float32)
    farthest0 = jnp.zeros((B, 1, 1), dtype=jnp.int32)
    acc0 = (
        jnp.zeros((B, GS, L), jnp.int32),
        jnp.zeros((B, GS, L), jnp.float32),
        jnp.zeros((B, GS, L), jnp.float32),
        jnp.zeros((B, GS, L), jnp.float32),
    )
    _, _, acc_idx, acc_cx, acc_cy, acc_cz = lax.fori_loop(
        0, GROUP, body, (distance0, farthest0) + acc0
    )
    idx_ref[...] = acc_idx
    cx_ref[...] = acc_cx
    cy_ref[...] = acc_cy
    cz_ref[...] = acc_cz


def _fps(xyz):
    B, N, _ = xyz.shape
    S = N // LANE
    GS = GROUP // LANE
    xs = xyz[:, :, 0].reshape(B, S, LANE)
    ys = xyz[:, :, 1].reshape(B, S, LANE)
    zs = xyz[:, :, 2].reshape(B, S, LANE)
    out_shapes = (
        jax.ShapeDtypeStruct((B, GS, LANE), jnp.int32),
        jax.ShapeDtypeStruct((B, GS, LANE), jnp.float32),
        jax.ShapeDtypeStruct((B, GS, LANE), jnp.float32),
        jax.ShapeDtypeStruct((B, GS, LANE), jnp.float32),
    )
    idx, cx, cy, cz = pl.pallas_call(
        _fps_body,
        out_shape=out_shapes,
    )(xs, ys, zs)
    return (
        idx.reshape(B, GROUP),
        cx.reshape(B, GROUP),
        cy.reshape(B, GROUP),
        cz.reshape(B, GROUP),
    )


def _dist_body(
    q_ref, p_ref, xs_ref, ys_ref, zs_ref,
    pp_ref, xsp_ref, ysp_ref, zsp_ref, out_ref, cm_ref,
):
    q = q_ref[0]
    p = p_ref[0]
    xs = xs_ref[...][0]
    ys = ys_ref[...][0]
    zs = zs_ref[...][0]
    dot = jnp.dot(q, p, preferred_element_type=jnp.float32)
    qx = q[:, 0:1]
    qy = q[:, 1:2]
    qz = q[:, 2:3]
    qn = (qx * qx + qy * qy) + qz * qz
    pn = (xs * xs + ys * ys) + zs * zs
    d = ((-2.0 * dot) + qn) + pn
    out_ref[0] = d
    pp = pp_ref[0]
    xsp = xsp_ref[...][0]
    ysp = ysp_ref[...][0]
    zsp = zsp_ref[...][0]
    dotp = jnp.dot(q, pp, preferred_element_type=jnp.float32)
    pnp = (xsp * xsp + ysp * ysp) + zsp * zsp
    dp = ((-2.0 * dotp) + qn) + pnp
    GB, N = d.shape
    NC = N // 16
    cm = dp[:, 0:NC]
    for kk in range(1, 16):
        cm = jnp.minimum(cm, dp[:, kk * NC : (kk + 1) * NC])
    cm_ref[0] = cm


def _dist(lc_xyz, xyz):
    B, G, _ = lc_xyz.shape
    N = xyz.shape[1]
    xyz_t = jnp.transpose(xyz, (0, 2, 1))
    xs = xyz[:, :, 0].reshape(B, 1, N)
    ys = xyz[:, :, 1].reshape(B, 1, N)
    zs = xyz[:, :, 2].reshape(B, 1, N)
    permute = lambda a: (
        a.reshape(B, a.shape[1], N // 16, 16)
        .transpose(0, 1, 3, 2)
        .reshape(B, a.shape[1], N)
    )
    xyz_p = permute(xyz_t)
    xsp, ysp, zsp = permute(xs), permute(ys), permute(zs)
    GB = 128
    qspec = pl.BlockSpec((1, GB, 3), lambda b, g: (b, g, 0))
    pspec = pl.BlockSpec((1, 3, N), lambda b, g: (b, 0, 0))
    cspec = pl.BlockSpec((1, 1, N), lambda b, g: (b, 0, 0))
    return pl.pallas_call(
        _dist_body,
        grid=(B, G // GB),
        in_specs=[qspec, pspec, cspec, cspec, cspec,
                  pspec, cspec, cspec, cspec],
        out_specs=(
            pl.BlockSpec((1, GB, N), lambda b, g: (b, g, 0)),
            pl.BlockSpec((1, GB, N // 16), lambda b, g: (b, g, 0)),
        ),
        out_shape=(
            jax.ShapeDtypeStruct((B, G, N), jnp.float32),
            jax.ShapeDtypeStruct((B, G, N // 16), jnp.float32),
        ),
    )(lc_xyz, xyz_t, xs, ys, zs, xyz_p, xsp, ysp, zsp)


def _vmerge(ad, ai, bd, bi):
    brd = lax.rev(bd, (0,))
    bri = lax.rev(bi, (0,))
    mle = (ad < brd) | ((ad == brd) & (ai <= bri))
    lod = jnp.where(mle, ad, brd)
    loi = jnp.where(mle, ai, bri)
    hid = jnp.where(mle, brd, ad)
    hii = jnp.where(mle, bri, ai)
    lod, loi = plsc.sort_key_val(lod, loi)
    hid, hii = plsc.sort_key_val(hid, hii)
    return lod, loi, hid, hii


def _topk_sc(dist, cm):
    B, G, N = dist.shape
    rows = B * G
    NC = N // 16
    rpw = rows // NWORK
    chunks = dist.reshape(rows * (N // LANE), LANE)
    cmf = cm.reshape(rows, NC)
    inf = jnp.float32(jnp.inf)

    mesh = plsc.VectorSubcoreMesh(core_axis_name="c", subcore_axis_name="s")

    @functools.partial(
        pl.kernel,
        out_type=jax.ShapeDtypeStruct((rows * KNN,), jnp.int32),
        mesh=mesh,
        scratch_types=[
            pltpu.VMEM((8, NC), jnp.float32),
            pltpu.VMEM((32,), jnp.int32),
            pltpu.VMEM((32,), jnp.int32),
            pltpu.VMEM((32, LANE), jnp.float32),
            pltpu.VMEM((CBUF + 1,), jnp.float32),
            pltpu.VMEM((CBUF + 1,), jnp.int32),
            pltpu.VMEM((rpw * KNN,), jnp.int32),
            pltpu.SemaphoreType.DMA,
        ],
        compiler_params=pltpu.CompilerParams(needs_layout_passes=False),
    )
    def k(chunks_hbm, cm_hbm, out_hbm, cmbuf, cidl, cidg, cbuf, cd, ci,
          obuf, sem):
        wid = lax.axis_index("s") * 2 + lax.axis_index("c")
        base_row = wid * rpw
        iota16 = lax.broadcasted_iota(jnp.int32, (16,), 0)

        def consolidate(carry):
            T, cnt, r0d, r0i, r1d, r1i = carry
            cd[pl.ds(cnt, 16)] = jnp.full((16,), inf, jnp.float32)
            nv = (cnt + 15) // 16

            def merge_one(i, rc):
                r0d, r0i, r1d, r1i = rc
                sd = cd[pl.ds(i * 16, 16)]
                si = ci[pl.ds(i * 16, 16)]
                sd, si = plsc.sort_key_val(sd, si)
                l0d, l0i, h0d, h0i = _vmerge(r0d, r0i, sd, si)
                l1d, l1i, _, _ = _vmerge(r1d, r1i, h0d, h0i)
                return l0d, l0i, l1d, l1i

            r0d, r0i, r1d, r1i = lax.fori_loop(
                0, nv, merge_one, (r0d, r0i, r1d, r1i)
            )
            return jnp.max(r1d), jnp.int32(0), r0d, r0i, r1d, r1i

        def append(v, idx16, carry):
            T, cnt, r0d, r0i, r1d, r1i = carry
            m = v < T
            cs = plsc.cumsum(jnp.where(m, 1, 0))
            pos = jnp.where(m, cnt + cs - 1, CBUF)
            plsc.store_scatter(cd, [pos], v)
            plsc.store_scatter(ci, [pos], idx16)
            cnt = cnt + jnp.max(cs)
            carry = (T, cnt, r0d, r0i, r1d, r1i)
            return lax.cond(cnt >= CMAX, consolidate, lambda c: c, carry)

        def row_fn(rl, _):
            row = base_row + rl
            rb = lax.rem(rl, 8)
            z16 = jnp.zeros((16,), jnp.int32)
            i16 = jnp.full((16,), inf, jnp.float32)
            carry0 = (inf, jnp.int32(0), i16, z16, i16, z16)

            def p1(j, carry):
                return append(cmbuf[rb, pl.ds(j * 16, 16)], iota16 + j * 16,
                              carry)

            carry = lax.fori_loop(0, NC // 16, p1, carry0)
            _, _, _, c0i, _, c1i = consolidate(carry)

            a0, _ = plsc.sort_key_val(c0i, c0i)
            a1, _ = plsc.sort_key_val(c1i, c1i)
            lo, _, hi, _ = _vmerge(a0, a0, a1, a1)
            gbase = row * (N // LANE)
            cidl[pl.ds(0, 16)] = lo
            cidl[pl.ds(16, 16)] = hi
            cidg[pl.ds(0, 16)] = lo // 8 + gbase
            cidg[pl.ds(16, 16)] = hi // 8 + gbase
            pltpu.async_copy(chunks_hbm.at[cidg], cbuf, sem).wait()

            def p2(v, carry):
                cid = plsc.load_gather(cidl, [jnp.full((16,), v, jnp.int32)])
                off = jnp.max(lax.rem(cid, 8))
                d16 = cbuf[v, pl.ds(off * 16, 16)]
                return append(d16, cid * 16 + iota16, carry)

            carry = lax.fori_loop(0, 32, p2, carry0)
            _, _, _, r0i, _, r1i = consolidate(carry)
            obuf[pl.ds(rl * KNN, 16)] = r0i
            obuf[pl.ds(rl * KNN + 16, 16)] = r1i
            return 0

        def group_fn(gI, _):
            pltpu.sync_copy(
                cm_hbm.at[pl.ds(base_row + gI * 8, 8)], cmbuf
            )
            lax.fori_loop(gI * 8, gI * 8 + 8, row_fn, 0)
            return 0

        lax.fori_loop(0, rpw // 8, group_fn, 0)
        pltpu.sync_copy(obuf, out_hbm.at[pl.ds(base_row * KNN, rpw * KNN)])

    return k(chunks, cmf).reshape(B, G, KNN)


def _gather_sc(x, xyz, rgb, fps_idx, knn_idx):
    B, N, D = x.shape
    G, K = GROUP, KNN
    rows = B * G
    rpw = rows // NWORK
    x_flat = x.reshape(B * N, D)
    planes = [xyz[:, :, i] for i in range(3)] + [rgb[:, :, i] for i in range(3)]
    fps_flat = fps_idx.reshape(rows)
    knn_flat = knn_idx.reshape(rows * K)

    f32 = jnp.float32
    out_type = (
        jax.ShapeDtypeStruct((rows, D), f32),
        jax.ShapeDtypeStruct((rows,), f32),
        jax.ShapeDtypeStruct((rows,), f32),
        jax.ShapeDtypeStruct((rows,), f32),
        jax.ShapeDtypeStruct((rows * K, D), f32),
        jax.ShapeDtypeStruct((rows * K,), f32),
        jax.ShapeDtypeStruct((rows * K,), f32),
        jax.ShapeDtypeStruct((rows * K,), f32),
        jax.ShapeDtypeStruct((rows * K,), f32),
        jax.ShapeDtypeStruct((rows * K,), f32),
        jax.ShapeDtypeStruct((rows * K,), f32),
    )
    mesh = plsc.VectorSubcoreMesh(core_axis_name="c", subcore_axis_name="s")

    @functools.partial(
        pl.kernel,
        out_type=out_type,
        mesh=mesh,
        scratch_types=[
            [pltpu.VMEM((N,), f32) for _ in range(6)],
            pltpu.VMEM((rpw,), jnp.int32),
            pltpu.VMEM((rpw * K,), jnp.int32),
            pltpu.VMEM((rpw * K,), jnp.int32),
            pltpu.VMEM((128,), jnp.int32),
            pltpu.VMEM((128, D), f32),
            [pltpu.VMEM((rpw,), f32) for _ in range(3)],
            [pltpu.VMEM((rpw * K,), f32) for _ in range(6)],
            pltpu.SemaphoreType.DMA,
        ],
        compiler_params=pltpu.CompilerParams(needs_layout_passes=False),
    )
    def k(
        xs_h, ys_h, zs_h, r_h, g_h, b_h, xf_h, fps_h, knn_h,
        lcx_h, lr_h, lg_h, lb_h, knnx_h, kxs_h, kys_h, kzs_h, kr_h, kg_h, kb_h,
        pv, fi, ki, kg, gbuf, rv, lst, kst, sem,
    ):
        wid = lax.axis_index("s") * 2 + lax.axis_index("c")
        b = wid // (NWORK // B)
        bbase = b * N
        plane_hs = (xs_h, ys_h, zs_h, r_h, g_h, b_h)
        for i in range(6):
            pltpu.sync_copy(plane_hs[i].at[b], pv[i])
        pltpu.sync_copy(fps_h.at[pl.ds(wid * rpw, rpw)], fi)
        pltpu.sync_copy(knn_h.at[pl.ds(wid * rpw * K, rpw * K)], ki)

        for j in range(rpw // 16):
            sl = pl.ds(j * 16, 16)
            idx16 = fi[sl]
            for i in range(3):
                lst[i][sl] = plsc.load_gather(pv[3 + i], [idx16])
            gbuf[sl] = idx16 + bbase
        pltpu.async_copy(xf_h.at[gbuf], rv, sem).wait()
        pltpu.sync_copy(rv, lcx_h.at[pl.ds(wid * rpw, rpw)])
        for i, oh in enumerate((lr_h, lg_h, lb_h)):
            pltpu.sync_copy(lst[i], oh.at[pl.ds(wid * rpw, rpw)])

        def plane_step(j, _):
            sl = pl.ds(j * 16, 16)
            idx16 = ki[sl]
            for i in range(6):
                kst[i][sl] = plsc.load_gather(pv[i], [idx16])
            kg[sl] = idx16 + bbase
            return 0

        lax.fori_loop(0, rpw * K // 16, plane_step, 0)
        outs = (kxs_h, kys_h, kzs_h, kr_h, kg_h, kb_h)
        for i in range(6):
            pltpu.sync_copy(kst[i], outs[i].at[pl.ds(wid * rpw * K, rpw * K)])

        def chunk_step(cI, _):
            for jj in range(8):
                gbuf[pl.ds(jj * 16, 16)] = kg[pl.ds(cI * 128 + jj * 16, 16)]
            pltpu.async_copy(xf_h.at[gbuf], rv, sem).wait()
            pltpu.sync_copy(
                rv, knnx_h.at[pl.ds(wid * rpw * K + cI * 128, 128)]
            )
            return 0

        lax.fori_loop(0, rpw * K // 128, chunk_step, 0)

    res = k(*planes, x_flat, fps_flat, knn_flat)
    lc_x = res[0].reshape(B, G, D)
    lc_rgb = jnp.stack(res[1:4], axis=-1).reshape(B, G, 3)
    knn_x = res[4].reshape(B, G, K, D)
    knn_xyz = jnp.stack(res[5:8], axis=-1).reshape(B, G, K, 3)
    knn_rgb = jnp.stack(res[8:11], axis=-1).reshape(B, G, K, 3)
    return lc_x, lc_rgb, knn_x, knn_xyz, knn_rgb


def kernel(xyz, x, rgb):
    fps_idx, cx, cy, cz = _fps(xyz)
    lc_xyz = jnp.stack([cx, cy, cz], axis=-1)

    dist, cm = _dist(lc_xyz, xyz)
    knn_idx = _topk_sc(dist, cm)

    lc_x, lc_rgb, knn_x, knn_xyz, knn_rgb = _gather_sc(
        x, xyz, rgb, fps_idx, knn_idx
    )
    return (lc_xyz, lc_x, lc_rgb, knn_xyz, knn_x, knn_rgb)
